# Initial kernel scaffold; baseline (speedup 1.0000x reference)
import jax
import jax.numpy as jnp
from jax.experimental import pallas as pl


def kernel(x, Wq, Wr):
    raise NotImplementedError("write your pallas kernel here")


# speedup vs baseline: 2.6354x; 2.6354x over previous
import functools

import jax
import jax.numpy as jnp
from jax import lax
from jax.experimental import pallas as pl
from jax.experimental.pallas import tpu as pltpu
from jax.experimental.pallas import tpu_sc as plsc

NUM_BUCKETS = 1000
EMBED_DIM = 64
BATCH = 4096
L = 26

NC = 2
NS = 16
DH = EMBED_DIM // NC
ROWS = BATCH // NS
CHUNK = 64
NCHUNK = ROWS // CHUNK
NGRP = CHUNK // 16
TW = NUM_BUCKETS * DH


def _qr_body(xf_hbm, wcat_hbm, out_hbm, wq_v, wr_v, xs_v, out_v):
    c = lax.axis_index("c")
    s = lax.axis_index("s")
    dbase = c * DH
    rbase = s * ROWS

    pltpu.sync_copy(wcat_hbm.at[pl.ds(c * TW, TW)], wq_v)
    pltpu.sync_copy(wcat_hbm.at[pl.ds((NC + c) * TW, TW)], wr_v)

    iota = lax.iota(jnp.int32, 16)
    iota_l = iota * L

    def chunk_body(ch, _):
        row0 = rbase + ch * CHUNK
        pltpu.sync_copy(xf_hbm.at[pl.ds(row0 * L, CHUNK * L)], xs_v)

        def grp_body(g, _):
            rowv = iota + g * 16
            xbase = iota_l + g * (16 * L)
            for dblk in range(DH // 16):
                accs = [jnp.zeros((16,), jnp.float32) for _ in range(16)]
                for l in range(L):
                    xv = plsc.load_gather(xs_v, [xbase + l])
                    qv = xv // NUM_BUCKETS
                    rv = xv - qv * NUM_BUCKETS
                    qi = qv * DH + dblk * 16
                    ri = rv * DH + dblk * 16
                    for dd in range(16):
                        gq = plsc.load_gather(wq_v, [qi + dd])
                        gr = plsc.load_gather(wr_v, [ri + dd])
                        accs[dd] = accs[dd] + gq * gr
                for dd in range(16):
                    col = jnp.full((16,), dblk * 16 + dd, jnp.int32)
                    plsc.store_scatter(out_v, [rowv, col], accs[dd])
            return 0

        lax.fori_loop(0, NGRP, grp_body, 0, unroll=False)
        pltpu.sync_copy(out_v, out_hbm.at[pl.ds(row0, CHUNK), pl.ds(dbase, DH)])
        return 0

    lax.fori_loop(0, NCHUNK, chunk_body, 0, unroll=False)


@jax.jit
def _qr_embedding(x, Wq, Wr):
    wcat = jnp.concatenate(
        [Wq[:, :DH].ravel(), Wq[:, DH:].ravel(),
         Wr[:, :DH].ravel(), Wr[:, DH:].ravel()]
    )
    xf = x.ravel()
    mesh = plsc.VectorSubcoreMesh(core_axis_name="c", subcore_axis_name="s")
    kern = functools.partial(
        pl.kernel,
        out_type=jax.ShapeDtypeStruct((BATCH, EMBED_DIM), jnp.float32),
        mesh=mesh,
        compiler_params=pltpu.CompilerParams(
            use_tc_tiling_on_sc=False, needs_layout_passes=False
        ),
        scratch_types=[
            pltpu.VMEM((TW,), jnp.float32),
            pltpu.VMEM((TW,), jnp.float32),
            pltpu.VMEM((CHUNK * L,), jnp.int32),
            pltpu.VMEM((CHUNK, DH), jnp.float32),
        ],
    )(_qr_body)
    return kern(xf, wcat)


def kernel(x, Wq, Wr):
    return _qr_embedding(x.astype(jnp.int32), Wq, Wr)


# speedup vs baseline: 3.7915x; 1.4387x over previous
import functools

import jax
import jax.numpy as jnp
from jax import lax
from jax.experimental import pallas as pl
from jax.experimental.pallas import tpu as pltpu
from jax.experimental.pallas import tpu_sc as plsc

NUM_BUCKETS = 1000
EMBED_DIM = 64
BATCH = 4096
L = 26

NC = 2
NS = 16
DH = EMBED_DIM // NC
DB = 16
ROWS = BATCH // NS
CHUNK = 64
NCHUNK = ROWS // CHUNK
NGRP = CHUNK // 16
TW = NUM_BUCKETS * DH


def _qr_body(xf_hbm, wcat_hbm, out_hbm, wq_v, wr_v, xs_v, qb_v, rb_v, out_t,
             out_v2):
    c = lax.axis_index("c")
    s = lax.axis_index("s")
    dbase = c * DH
    rbase = s * ROWS

    pltpu.sync_copy(wcat_hbm.at[pl.ds(c * TW, TW)], wq_v)
    pltpu.sync_copy(wcat_hbm.at[pl.ds((NC + c) * TW, TW)], wr_v)

    iota = lax.iota(jnp.int32, 16)
    iota_l = iota * L
    iota_c = iota * CHUNK

    def chunk_body(ch, _):
        row0 = rbase + ch * CHUNK
        pltpu.sync_copy(xf_hbm.at[pl.ds(row0 * L, CHUNK * L)], xs_v)

        def grp_body(g, _):
            goff = g * 16
            xbase = iota_l + g * (16 * L)

            @plsc.parallel_loop(0, L, unroll=2)
            def _prepass(l):
                xv = plsc.load_gather(xs_v, [xbase + l])
                xf32 = xv.astype(jnp.float32)
                qv = (xf32 * jnp.float32(0.001)).astype(jnp.int32)
                rv = xv - qv * NUM_BUCKETS
                qb_v[pl.ds(l * 16, 16)] = qv * DH
                rb_v[pl.ds(l * 16, 16)] = rv * DH

            for blk in range(DH // DB):
                zeros = tuple(jnp.zeros((16,), jnp.float32) for _ in range(DB))

                @plsc.parallel_loop(0, L, unroll=2, carry=zeros)
                def _gather(l, accs):
                    qb = qb_v[pl.ds(l * 16, 16)]
                    rb = rb_v[pl.ds(l * 16, 16)]
                    new = []
                    for dd in range(DB):
                        d = blk * DB + dd
                        gq = plsc.load_gather(wq_v, [qb + d])
                        gr = plsc.load_gather(wr_v, [rb + d])
                        new.append(accs[dd] + gq * gr)
                    return tuple(new)

                for dd in range(DB):
                    out_t[blk * DB + dd, pl.ds(goff, 16)] = _gather[dd]
            return 0

        lax.fori_loop(0, NGRP, grp_body, 0, unroll=False)

        def rep_body(row, _):
            for k in range(DH // 16):
                out_v2[row, pl.ds(k * 16, 16)] = plsc.load_gather(
                    out_t, [iota + k * 16, jnp.full((16,), row, jnp.int32)]
                )
            return 0

        lax.fori_loop(0, CHUNK, rep_body, 0, unroll=False)
        pltpu.sync_copy(out_v2, out_hbm.at[pl.ds(row0, CHUNK), pl.ds(dbase, DH)])
        return 0

    lax.fori_loop(0, NCHUNK, chunk_body, 0, unroll=False)


@jax.jit
def _qr_embedding(x, Wq, Wr):
    wcat = jnp.concatenate(
        [Wq[:, :DH].ravel(), Wq[:, DH:].ravel(),
         Wr[:, :DH].ravel(), Wr[:, DH:].ravel()]
    )
    xf = x.ravel()
    mesh = plsc.VectorSubcoreMesh(core_axis_name="c", subcore_axis_name="s")
    kern = functools.partial(
        pl.kernel,
        out_type=jax.ShapeDtypeStruct((BATCH, EMBED_DIM), jnp.float32),
        mesh=mesh,
        compiler_params=pltpu.CompilerParams(
            use_tc_tiling_on_sc=False, needs_layout_passes=False
        ),
        scratch_types=[
            pltpu.VMEM((TW,), jnp.float32),
            pltpu.VMEM((TW,), jnp.float32),
            pltpu.VMEM((CHUNK * L,), jnp.int32),
            pltpu.VMEM((L * 16,), jnp.int32),
            pltpu.VMEM((L * 16,), jnp.int32),
            pltpu.VMEM((DH, CHUNK), jnp.float32),
            pltpu.VMEM((CHUNK, DH), jnp.float32),
        ],
    )(_qr_body)
    return kern(xf, wcat)


def kernel(x, Wq, Wr):
    return _qr_embedding(x.astype(jnp.int32), Wq, Wr)


# speedup vs baseline: 11.8109x; 3.1151x over previous
import functools

import jax
import jax.numpy as jnp
from jax import lax
from jax.experimental import pallas as pl
from jax.experimental.pallas import tpu as pltpu
from jax.experimental.pallas import tpu_sc as plsc

NUM_BUCKETS = 1000
EMBED_DIM = 64
BATCH = 4096
L = 26

NC = 2
NS = 16
DH = EMBED_DIM // NC
DB = 16
ROWS = BATCH // NS
CHUNK = 64
NCHUNK = ROWS // CHUNK
NGRP = CHUNK // 16
TS = DH + 1
TW = NUM_BUCKETS * TS
OS = CHUNK + 1


def _qr_body(xf_hbm, wcat_hbm, out_hbm, wq_v, wr_v, xs_v, qb_v, rb_v, out_t,
             out_v2):
    c = lax.axis_index("c")
    s = lax.axis_index("s")
    dbase = c * DH
    rbase = s * ROWS

    pltpu.sync_copy(wcat_hbm.at[pl.ds(c * TW, TW)], wq_v)
    pltpu.sync_copy(wcat_hbm.at[pl.ds((NC + c) * TW, TW)], wr_v)

    iota = lax.iota(jnp.int32, 16)
    iota_l = iota * L


    def chunk_body(ch, _):
        row0 = rbase + ch * CHUNK
        pltpu.sync_copy(xf_hbm.at[pl.ds(row0 * L, CHUNK * L)], xs_v)

        def grp_body(g, _):
            goff = g * 16
            xbase = iota_l + g * (16 * L)

            @plsc.parallel_loop(0, L, unroll=2)
            def _prepass(l):
                xv = plsc.load_gather(xs_v, [xbase + l])
                xf32 = xv.astype(jnp.float32)
                qv = (xf32 * jnp.float32(0.001)).astype(jnp.int32)
                rv = xv - qv * NUM_BUCKETS
                qb_v[pl.ds(l * 16, 16)] = qv * TS
                rb_v[pl.ds(l * 16, 16)] = rv * TS

            for blk in range(DH // DB):
                zeros = tuple(jnp.zeros((16,), jnp.float32) for _ in range(DB))

                @plsc.parallel_loop(0, L, unroll=2, carry=zeros)
                def _gather(l, accs):
                    qb = qb_v[pl.ds(l * 16, 16)]
                    rb = rb_v[pl.ds(l * 16, 16)]
                    new = []
                    for dd in range(DB):
                        d = blk * DB + dd
                        gq = plsc.load_gather(wq_v, [qb + d])
                        gr = plsc.load_gather(wr_v, [rb + d])
                        new.append(accs[dd] + gq * gr)
                    return tuple(new)

                for dd in range(DB):
                    out_t[blk * DB + dd, pl.ds(goff, 16)] = _gather[dd]
            return 0

        lax.fori_loop(0, NGRP, grp_body, 0, unroll=False)

        def rep_body(row, _):
            for k in range(DH // 16):
                out_v2[row, pl.ds(k * 16, 16)] = plsc.load_gather(
                    out_t, [iota + k * 16, jnp.full((16,), row, jnp.int32)]
                )
            return 0

        lax.fori_loop(0, CHUNK, rep_body, 0, unroll=False)
        pltpu.sync_copy(out_v2, out_hbm.at[pl.ds(row0, CHUNK), pl.ds(dbase, DH)])
        return 0

    lax.fori_loop(0, NCHUNK, chunk_body, 0, unroll=False)


@jax.jit
def _qr_embedding(x, Wq, Wr):
    def _slice_pad(W, lo):
        return jnp.pad(W[:, lo:lo + DH], ((0, 0), (0, TS - DH))).ravel()

    wcat = jnp.concatenate(
        [_slice_pad(Wq, 0), _slice_pad(Wq, DH),
         _slice_pad(Wr, 0), _slice_pad(Wr, DH)]
    )
    xf = x.ravel()
    mesh = plsc.VectorSubcoreMesh(core_axis_name="c", subcore_axis_name="s")
    kern = functools.partial(
        pl.kernel,
        out_type=jax.ShapeDtypeStruct((BATCH, EMBED_DIM), jnp.float32),
        mesh=mesh,
        compiler_params=pltpu.CompilerParams(
            use_tc_tiling_on_sc=False, needs_layout_passes=False
        ),
        scratch_types=[
            pltpu.VMEM((TW,), jnp.float32),
            pltpu.VMEM((TW,), jnp.float32),
            pltpu.VMEM((CHUNK * L,), jnp.int32),
            pltpu.VMEM((L * 16,), jnp.int32),
            pltpu.VMEM((L * 16,), jnp.int32),
            pltpu.VMEM((DH, OS), jnp.float32),
            pltpu.VMEM((CHUNK, DH), jnp.float32),
        ],
    )(_qr_body)
    return kern(xf, wcat)


def kernel(x, Wq, Wr):
    return _qr_embedding(x.astype(jnp.int32), Wq, Wr)


# speedup vs baseline: 12.6879x; 1.0743x over previous
import functools

import jax
import jax.numpy as jnp
from jax import lax
from jax.experimental import pallas as pl
from jax.experimental.pallas import tpu as pltpu
from jax.experimental.pallas import tpu_sc as plsc

NUM_BUCKETS = 1000
EMBED_DIM = 64
BATCH = 4096
L = 26

NC = 2
NS = 16
DH = EMBED_DIM // NC
DB = 16
ROWS = BATCH // NS
CHUNK = 64
NCHUNK = ROWS // CHUNK
NGRP = CHUNK // 16
TS = DH + 1
TW = NUM_BUCKETS * TS


def _qr_body(xf_hbm, wcat_hbm, out_hbm, wq_v, wr_v, xs_v, qb_v, rb_v,
             out_v2):
    c = lax.axis_index("c")
    s = lax.axis_index("s")
    dbase = c * DH
    rbase = s * ROWS

    pltpu.sync_copy(wcat_hbm.at[pl.ds(c * TW, TW)], wq_v)
    pltpu.sync_copy(wcat_hbm.at[pl.ds((NC + c) * TW, TW)], wr_v)

    iota = lax.iota(jnp.int32, 16)
    iota_l = iota * L


    def chunk_body(ch, _):
        row0 = rbase + ch * CHUNK
        pltpu.sync_copy(xf_hbm.at[pl.ds(row0 * L, CHUNK * L)], xs_v)

        def grp_body(g, _):
            goff = g * 16
            xbase = iota_l + g * (16 * L)

            @plsc.parallel_loop(0, L, unroll=2)
            def _prepass(l):
                xv = plsc.load_gather(xs_v, [xbase + l])
                xf32 = xv.astype(jnp.float32)
                qv = (xf32 * jnp.float32(0.001)).astype(jnp.int32)
                rv = xv - qv * NUM_BUCKETS
                qb_v[pl.ds(l * 16, 16)] = qv * TS
                rb_v[pl.ds(l * 16, 16)] = rv * TS

            for blk in range(DH // DB):
                dofs = iota + blk * DB
                zeros = tuple(jnp.zeros((16,), jnp.float32) for _ in range(16))

                @plsc.parallel_loop(0, L, unroll=2, carry=zeros)
                def _gather(l, accs):
                    qb = qb_v[pl.ds(l * 16, 16)]
                    rb = rb_v[pl.ds(l * 16, 16)]
                    new = []
                    for b in range(16):
                        lane = jnp.full((16,), b, jnp.int32)
                        qs = jnp.take_along_axis(
                            qb, lane, axis=0, mode="promise_in_bounds")
                        rs = jnp.take_along_axis(
                            rb, lane, axis=0, mode="promise_in_bounds")
                        gq = plsc.load_gather(wq_v, [qs + dofs])
                        gr = plsc.load_gather(wr_v, [rs + dofs])
                        new.append(accs[b] + gq * gr)
                    return tuple(new)

                for b in range(16):
                    out_v2[goff + b, pl.ds(blk * DB, DB)] = _gather[b]
            return 0

        lax.fori_loop(0, NGRP, grp_body, 0, unroll=False)
        pltpu.sync_copy(out_v2, out_hbm.at[pl.ds(row0, CHUNK), pl.ds(dbase, DH)])
        return 0

    lax.fori_loop(0, NCHUNK, chunk_body, 0, unroll=False)


@jax.jit
def _qr_embedding(x, Wq, Wr):
    def _slice_pad(W, lo):
        return jnp.pad(W[:, lo:lo + DH], ((0, 0), (0, TS - DH))).ravel()

    wcat = jnp.concatenate(
        [_slice_pad(Wq, 0), _slice_pad(Wq, DH),
         _slice_pad(Wr, 0), _slice_pad(Wr, DH)]
    )
    xf = x.ravel()
    mesh = plsc.VectorSubcoreMesh(core_axis_name="c", subcore_axis_name="s")
    kern = functools.partial(
        pl.kernel,
        out_type=jax.ShapeDtypeStruct((BATCH, EMBED_DIM), jnp.float32),
        mesh=mesh,
        compiler_params=pltpu.CompilerParams(
            use_tc_tiling_on_sc=False, needs_layout_passes=False
        ),
        scratch_types=[
            pltpu.VMEM((TW,), jnp.float32),
            pltpu.VMEM((TW,), jnp.float32),
            pltpu.VMEM((CHUNK * L,), jnp.int32),
            pltpu.VMEM((L * 16,), jnp.int32),
            pltpu.VMEM((L * 16,), jnp.int32),
            pltpu.VMEM((CHUNK, DH), jnp.float32),
        ],
    )(_qr_body)
    return kern(xf, wcat)


def kernel(x, Wq, Wr):
    return _qr_embedding(x.astype(jnp.int32), Wq, Wr)
